# baseline (speedup 1.0000x reference)
import jax
import jax.numpy as jnp
from jax import lax
from jax.experimental import pallas as pl
from jax.experimental.pallas import tpu as pltpu
from jax.experimental.pallas import tpu_sc as plsc

_VOCAB = 1000000
_DIM = 64
_BATCH = 4096
_BAG = 50

_C = 8192
_HC = _C // 2
_HBITS = _HC.bit_length() - 1
_Q = _C // 8
_QBITS = _Q.bit_length() - 1
_TC_GRID = (_VOCAB + _C - 1) // _C
_VPAD = _TC_GRID * _C
_REC = 32

_NC = 2
_NS = 16
_NW = _NC * _NS
_BPW = _BATCH // _NW
_NBUF = 4
_L = 16
_NV = _DIM // _L
_NIB = (0, 4, 2, 6)


def _tc_body(w_ref, out_ref):
    x = w_ref[...]
    m = jnp.max(jnp.abs(x), axis=0)
    scale = jnp.where(m == 0.0, 1.0, m / 7.0)
    q = jnp.round(x / scale[None, :])
    qi = q.astype(jnp.int32)
    z = jnp.concatenate([qi[:, :_HC], qi[:, _HC:]], axis=0) & 15
    a = z[0:64] | (z[64:128] << 4)
    b = a[0:32] | (a[32:64] << 8)
    c = b[0:16] | (b[16:32] << 16)
    sbits = lax.bitcast_convert_type(scale, jnp.int32)
    sa = sbits[None, :_HC]
    sb = sbits[None, _HC:]
    pads = jnp.zeros((_REC - 18, _Q), jnp.int32)
    pieces = []
    for t in range(4):
        sl = slice(t * _Q, (t + 1) * _Q)
        pieces += [c[:, sl], sa[:, sl], sb[:, sl], pads]
    u = jnp.concatenate(pieces, axis=0)
    out_ref[...] = u.T


_tc_quant = pl.pallas_call(
    _tc_body,
    grid=(_TC_GRID,),
    in_specs=[pl.BlockSpec((_DIM, _C), lambda k: (0, k))],
    out_specs=pl.BlockSpec((_C // 8, 4 * _REC), lambda k: (k, 0)),
    out_shape=jax.ShapeDtypeStruct((_VPAD // 8, 4 * _REC), jnp.int32),
)


def _sc_body(table_hbm, idx_hbm, cat_hbm, out_hbm, idx_v, idx_t, cat_v,
             rows_v, stage_v, sem):
    wid = lax.axis_index("s") * _NC + lax.axis_index("c")
    base = wid * _BPW

    pltpu.sync_copy(idx_hbm.at[pl.ds(base, _BPW)], idx_v)
    pltpu.sync_copy(cat_hbm.at[pl.ds(base, _BPW)], cat_v)

    def perm_body(r, carry):
        for off in (0, 16, 32, 34):
            v = idx_v[r, pl.ds(off, _L)]
            j = v & (_HC - 1)
            p = (lax.shift_right_logical(v & ~(_C - 1), 1)
                 | ((j & (_Q - 1)) << 2)
                 | lax.shift_right_logical(j, _QBITS))
            idx_t[r, pl.ds(off, _L)] = p
        return carry

    lax.fori_loop(0, _BPW, perm_body, 0, unroll=4)

    for s in range(_NBUF):
        pltpu.async_copy(table_hbm.at[idx_t.at[s]], rows_v.at[s], sem.at[s])

    _WINDOWS = ((0, range(0, 16)), (16, range(0, 16)), (32, range(0, 16)),
                (34, range(14, 16)))

    def bag_body(g, carry):
        s = g & (_NBUF - 1)
        pltpu.make_async_copy(
            table_hbm.at[idx_t.at[g]], rows_v.at[s], sem.at[s]).wait()
        acc = [jnp.zeros((_L,), jnp.float32) for _ in range(_NV)]
        for woff, lanes in _WINDOWS:
            hv = lax.shift_right_logical(
                idx_v[g, pl.ds(woff, _L)], _HBITS) & 1
            for lane in lanes:
                r = woff + lane
                w0 = rows_v[s, r, pl.ds(0, _L)]
                sv = lax.bitcast_convert_type(
                    rows_v[s, r, pl.ds(_L, _L)], jnp.float32)
                h = hv[lane]
                scale = jnp.where(h == 0, sv[0], sv[1])
                hs = 4 * h
                for k in range(_NV):
                    qk = lax.shift_right_arithmetic(
                        lax.shift_left(w0, (28 - 4 * _NIB[k]) - hs), 28)
                    acc[k] = acc[k] + qk.astype(jnp.float32) * scale
        gn = g + _NBUF

        @pl.when(gn < _BPW)
        def _():
            pltpu.async_copy(
                table_hbm.at[idx_t.at[gn]], rows_v.at[s], sem.at[s])

        for k in range(_NV):
            stage_v[g, pl.ds(k * _L, _L)] = acc[k]
            stage_v[g, pl.ds(_DIM + k * _L, _L)] = acc[k]
            stage_v[g, pl.ds(2 * _DIM + k * _L, _L)] = \
                cat_v[g, pl.ds(k * _L, _L)]
        return carry

    lax.fori_loop(0, _BPW, bag_body, 0)
    pltpu.sync_copy(stage_v, out_hbm.at[pl.ds(base, _BPW)])


import functools


@functools.cache
def _sc_embed():
    return pl.kernel(
        _sc_body,
        out_type=jax.ShapeDtypeStruct((_BATCH, 3 * _DIM), jnp.float32),
        mesh=plsc.VectorSubcoreMesh(core_axis_name="c",
                                    subcore_axis_name="s",
                                    num_cores=_NC, num_subcores=_NS),
        compiler_params=pltpu.CompilerParams(use_tc_tiling_on_sc=False),
        scratch_types=[
            pltpu.VMEM((_BPW, _BAG), jnp.int32),
            pltpu.VMEM((_BPW, _BAG), jnp.int32),
            pltpu.VMEM((_BPW, _DIM), jnp.float32),
            pltpu.VMEM((_NBUF, _BAG, _REC), jnp.int32),
            pltpu.VMEM((_BPW, 3 * _DIM), jnp.float32),
            pltpu.SemaphoreType.DMA((_NBUF,)),
        ],
    )


def kernel(weights, indices, offsets, cat_input, output_dtype):
    del offsets, output_dtype
    packed = _tc_quant(weights.T).reshape(_VPAD // 2, _REC)
    idx2d = indices.reshape(_BATCH, _BAG)
    return _sc_embed()(packed, idx2d, cat_input)


# speedup vs baseline: 1.1815x; 1.1815x over previous
import jax
import jax.numpy as jnp
from jax import lax
from jax.experimental import pallas as pl
from jax.experimental.pallas import tpu as pltpu
from jax.experimental.pallas import tpu_sc as plsc

_VOCAB = 1000000
_DIM = 64
_BATCH = 4096
_BAG = 50

_C = 16384
_HC = _C // 2
_HBITS = _HC.bit_length() - 1
_Q = _C // 8
_QBITS = _Q.bit_length() - 1
_TC_GRID = (_VOCAB + _C - 1) // _C
_VPAD = _TC_GRID * _C
_REC = 32

_NC = 2
_NS = 16
_NW = _NC * _NS
_BPW = _BATCH // _NW
_NBUF = 4
_L = 16
_NV = _DIM // _L
_NIB = (0, 4, 2, 6)


def _tc_body(w_ref, out_ref):
    x = w_ref[...]
    m = jnp.max(jnp.abs(x), axis=0)
    scale = jnp.where(m == 0.0, 1.0, m / 7.0)
    q = jnp.round(x / scale[None, :])
    qi = q.astype(jnp.int32)
    z = jnp.concatenate([qi[:, :_HC], qi[:, _HC:]], axis=0) & 15
    a = z[0:64] | (z[64:128] << 4)
    b = a[0:32] | (a[32:64] << 8)
    c = b[0:16] | (b[16:32] << 16)
    sbits = lax.bitcast_convert_type(scale, jnp.int32)
    sa = sbits[None, :_HC]
    sb = sbits[None, _HC:]
    pads = jnp.zeros((_REC - 18, _Q), jnp.int32)
    pieces = []
    for t in range(4):
        sl = slice(t * _Q, (t + 1) * _Q)
        pieces += [c[:, sl], sa[:, sl], sb[:, sl], pads]
    u = jnp.concatenate(pieces, axis=0)
    out_ref[...] = u.T


_tc_quant = pl.pallas_call(
    _tc_body,
    grid=(_TC_GRID,),
    in_specs=[pl.BlockSpec((_DIM, _C), lambda k: (0, k))],
    out_specs=pl.BlockSpec((_C // 8, 4 * _REC), lambda k: (k, 0)),
    out_shape=jax.ShapeDtypeStruct((_VPAD // 8, 4 * _REC), jnp.int32),
)


def _sc_body(table_hbm, idx_hbm, cat_hbm, out_hbm, idx_v, idx_t, cat_v,
             rows_v, stage_v, sem):
    wid = lax.axis_index("s") * _NC + lax.axis_index("c")
    base = wid * _BPW

    pltpu.sync_copy(idx_hbm.at[pl.ds(base, _BPW)], idx_v)
    pltpu.sync_copy(cat_hbm.at[pl.ds(base, _BPW)], cat_v)

    def perm_body(r, carry):
        for off in (0, 16, 32, 34):
            v = idx_v[r, pl.ds(off, _L)]
            j = v & (_HC - 1)
            p = (lax.shift_right_logical(v & ~(_C - 1), 1)
                 | ((j & (_Q - 1)) << 2)
                 | lax.shift_right_logical(j, _QBITS))
            idx_t[r, pl.ds(off, _L)] = p
        return carry

    lax.fori_loop(0, _BPW, perm_body, 0, unroll=4)

    for s in range(_NBUF):
        pltpu.async_copy(table_hbm.at[idx_t.at[s]], rows_v.at[s], sem.at[s])

    _WINDOWS = ((0, range(0, 16)), (16, range(0, 16)), (32, range(0, 16)),
                (34, range(14, 16)))

    def bag_body(g, carry):
        s = g & (_NBUF - 1)
        pltpu.make_async_copy(
            table_hbm.at[idx_t.at[g]], rows_v.at[s], sem.at[s]).wait()
        acc = [jnp.zeros((_L,), jnp.float32) for _ in range(_NV)]
        for woff, lanes in _WINDOWS:
            hv = lax.shift_right_logical(
                idx_v[g, pl.ds(woff, _L)], _HBITS) & 1
            for lane in lanes:
                r = woff + lane
                w0 = rows_v[s, r, pl.ds(0, _L)]
                sv = lax.bitcast_convert_type(
                    rows_v[s, r, pl.ds(_L, _L)], jnp.float32)
                h = hv[lane]
                scale = jnp.where(h == 0, sv[0], sv[1])
                hs = 4 * h
                for k in range(_NV):
                    qk = lax.shift_right_arithmetic(
                        lax.shift_left(w0, (28 - 4 * _NIB[k]) - hs), 28)
                    acc[k] = acc[k] + qk.astype(jnp.float32) * scale
        gn = g + _NBUF

        @pl.when(gn < _BPW)
        def _():
            pltpu.async_copy(
                table_hbm.at[idx_t.at[gn]], rows_v.at[s], sem.at[s])

        for k in range(_NV):
            stage_v[g, pl.ds(k * _L, _L)] = acc[k]
            stage_v[g, pl.ds(_DIM + k * _L, _L)] = acc[k]
            stage_v[g, pl.ds(2 * _DIM + k * _L, _L)] = \
                cat_v[g, pl.ds(k * _L, _L)]
        return carry

    lax.fori_loop(0, _BPW, bag_body, 0)
    pltpu.sync_copy(stage_v, out_hbm.at[pl.ds(base, _BPW)])


import functools


@functools.cache
def _sc_embed():
    return pl.kernel(
        _sc_body,
        out_type=jax.ShapeDtypeStruct((_BATCH, 3 * _DIM), jnp.float32),
        mesh=plsc.VectorSubcoreMesh(core_axis_name="c",
                                    subcore_axis_name="s",
                                    num_cores=_NC, num_subcores=_NS),
        compiler_params=pltpu.CompilerParams(use_tc_tiling_on_sc=False),
        scratch_types=[
            pltpu.VMEM((_BPW, _BAG), jnp.int32),
            pltpu.VMEM((_BPW, _BAG), jnp.int32),
            pltpu.VMEM((_BPW, _DIM), jnp.float32),
            pltpu.VMEM((_NBUF, _BAG, _REC), jnp.int32),
            pltpu.VMEM((_BPW, 3 * _DIM), jnp.float32),
            pltpu.SemaphoreType.DMA((_NBUF,)),
        ],
    )


def kernel(weights, indices, offsets, cat_input, output_dtype):
    del offsets, output_dtype
    packed = _tc_quant(weights.T).reshape(_VPAD // 2, _REC)
    idx2d = indices.reshape(_BATCH, _BAG)
    return _sc_embed()(packed, idx2d, cat_input)


# speedup vs baseline: 1.2980x; 1.0986x over previous
import jax
import jax.numpy as jnp
from jax import lax
from jax.experimental import pallas as pl
from jax.experimental.pallas import tpu as pltpu
from jax.experimental.pallas import tpu_sc as plsc

_VOCAB = 1000000
_DIM = 64
_BATCH = 4096
_BAG = 50

_C = 32768
_HC = _C // 2
_HBITS = _HC.bit_length() - 1
_Q = _C // 8
_QBITS = _Q.bit_length() - 1
_TC_GRID = (_VOCAB + _C - 1) // _C
_VPAD = _TC_GRID * _C
_REC = 32

_NC = 2
_NS = 16
_NW = _NC * _NS
_BPW = _BATCH // _NW
_NBUF = 4
_L = 16
_NV = _DIM // _L
_NIB = (0, 4, 2, 6)


def _tc_body(w_ref, out_ref):
    x = w_ref[...]
    m = jnp.max(jnp.abs(x), axis=0)
    scale = jnp.where(m == 0.0, 1.0, m / 7.0)
    q = jnp.round(x / scale[None, :])
    qi = q.astype(jnp.int32)
    z = jnp.concatenate([qi[:, :_HC], qi[:, _HC:]], axis=0) & 15
    a = z[0:64] | (z[64:128] << 4)
    b = a[0:32] | (a[32:64] << 8)
    c = b[0:16] | (b[16:32] << 16)
    sbits = lax.bitcast_convert_type(scale, jnp.int32)
    sa = sbits[None, :_HC]
    sb = sbits[None, _HC:]
    pads = jnp.zeros((_REC - 18, _Q), jnp.int32)
    pieces = []
    for t in range(4):
        sl = slice(t * _Q, (t + 1) * _Q)
        pieces += [c[:, sl], sa[:, sl], sb[:, sl], pads]
    u = jnp.concatenate(pieces, axis=0)
    out_ref[...] = u.T


_tc_quant = pl.pallas_call(
    _tc_body,
    grid=(_TC_GRID,),
    in_specs=[pl.BlockSpec((_DIM, _C), lambda k: (0, k))],
    out_specs=pl.BlockSpec((_C // 8, 4 * _REC), lambda k: (k, 0)),
    out_shape=jax.ShapeDtypeStruct((_VPAD // 8, 4 * _REC), jnp.int32),
)


def _sc_body(table_hbm, idx_hbm, cat_hbm, out_hbm, idx_v, idx_t, cat_v,
             rows_v, stage_v, sem):
    wid = lax.axis_index("s") * _NC + lax.axis_index("c")
    base = wid * _BPW

    pltpu.sync_copy(idx_hbm.at[pl.ds(base, _BPW)], idx_v)
    pltpu.sync_copy(cat_hbm.at[pl.ds(base, _BPW)], cat_v)

    def perm_body(r, carry):
        for off in (0, 16, 32, 34):
            v = idx_v[r, pl.ds(off, _L)]
            j = v & (_HC - 1)
            p = (lax.shift_right_logical(v & ~(_C - 1), 1)
                 | ((j & (_Q - 1)) << 2)
                 | lax.shift_right_logical(j, _QBITS))
            idx_t[r, pl.ds(off, _L)] = p
        return carry

    lax.fori_loop(0, _BPW, perm_body, 0, unroll=4)

    for s in range(_NBUF):
        pltpu.async_copy(table_hbm.at[idx_t.at[s]], rows_v.at[s], sem.at[s])

    _WINDOWS = ((0, range(0, 16)), (16, range(0, 16)), (32, range(0, 16)),
                (34, range(14, 16)))

    def bag_body(g, carry):
        s = g & (_NBUF - 1)
        pltpu.make_async_copy(
            table_hbm.at[idx_t.at[g]], rows_v.at[s], sem.at[s]).wait()
        acc = [jnp.zeros((_L,), jnp.float32) for _ in range(_NV)]
        for woff, lanes in _WINDOWS:
            hv = lax.shift_right_logical(
                idx_v[g, pl.ds(woff, _L)], _HBITS) & 1
            for lane in lanes:
                r = woff + lane
                w0 = rows_v[s, r, pl.ds(0, _L)]
                sv = lax.bitcast_convert_type(
                    rows_v[s, r, pl.ds(_L, _L)], jnp.float32)
                h = hv[lane]
                scale = jnp.where(h == 0, sv[0], sv[1])
                hs = 4 * h
                for k in range(_NV):
                    qk = lax.shift_right_arithmetic(
                        lax.shift_left(w0, (28 - 4 * _NIB[k]) - hs), 28)
                    acc[k] = acc[k] + qk.astype(jnp.float32) * scale
        gn = g + _NBUF

        @pl.when(gn < _BPW)
        def _():
            pltpu.async_copy(
                table_hbm.at[idx_t.at[gn]], rows_v.at[s], sem.at[s])

        for k in range(_NV):
            stage_v[g, pl.ds(k * _L, _L)] = acc[k]
            stage_v[g, pl.ds(_DIM + k * _L, _L)] = acc[k]
            stage_v[g, pl.ds(2 * _DIM + k * _L, _L)] = \
                cat_v[g, pl.ds(k * _L, _L)]
        return carry

    lax.fori_loop(0, _BPW, bag_body, 0)
    pltpu.sync_copy(stage_v, out_hbm.at[pl.ds(base, _BPW)])


import functools


@functools.cache
def _sc_embed():
    return pl.kernel(
        _sc_body,
        out_type=jax.ShapeDtypeStruct((_BATCH, 3 * _DIM), jnp.float32),
        mesh=plsc.VectorSubcoreMesh(core_axis_name="c",
                                    subcore_axis_name="s",
                                    num_cores=_NC, num_subcores=_NS),
        compiler_params=pltpu.CompilerParams(use_tc_tiling_on_sc=False),
        scratch_types=[
            pltpu.VMEM((_BPW, _BAG), jnp.int32),
            pltpu.VMEM((_BPW, _BAG), jnp.int32),
            pltpu.VMEM((_BPW, _DIM), jnp.float32),
            pltpu.VMEM((_NBUF, _BAG, _REC), jnp.int32),
            pltpu.VMEM((_BPW, 3 * _DIM), jnp.float32),
            pltpu.SemaphoreType.DMA((_NBUF,)),
        ],
    )


def kernel(weights, indices, offsets, cat_input, output_dtype):
    del offsets, output_dtype
    packed = _tc_quant(weights.T).reshape(_VPAD // 2, _REC)
    idx2d = indices.reshape(_BATCH, _BAG)
    return _sc_embed()(packed, idx2d, cat_input)
